# baseline (speedup 1.0000x reference)
import functools

import jax
import jax.numpy as jnp
from jax import lax
from jax.experimental import pallas as pl
from jax.experimental.pallas import tpu as pltpu
from jax.experimental.pallas import tpu_sc as plsc

N = 10000
NP = 10240
E = 320000
EP = 327680
K1 = 7000
NW = 32
NC = 2
NS = 16
CHUNKS = 80
GK = 8
C_SLOW = 48
C_FAST = 112
RPT = NP // NS

_C0 = (K1 + 2) // 3
_C1 = (K1 + 1) // 3
_C2 = K1 // 3


def _gelu(v):
    return 0.5 * v * (1.0 + lax.erf(v * 0.7071067811865476))


def _sort_key(score):
    s = lax.bitcast_convert_type(score, jnp.int32)
    return jnp.where(s >= 0, s, s ^ jnp.int32(0x7FFFFFFF))


def _tc1_body(x_ref, w_ref, b_ref, o_ref):
    h = jnp.dot(x_ref[...], w_ref[...],
                preferred_element_type=jnp.float32) + b_ref[...]
    o_ref[...] = jnp.zeros((NP, 16), jnp.float32)
    o_ref[:N, 0:4] = _gelu(h)


def _tc1(x, w_root1, b_rel1):
    return pl.pallas_call(
        _tc1_body,
        out_shape=jax.ShapeDtypeStruct((NP, 16), jnp.float32),
    )(x, w_root1, b_rel1)


def _tc2_body(aggp_ref, h1_ref, wrel_ref, brel_ref, wroot_ref, pw_ref,
              h_ref, score_ref, key_ref):
    agg = aggp_ref[:NP, 0:4] + aggp_ref[NP:, 0:4]
    h = (jnp.dot(agg, wrel_ref[...], preferred_element_type=jnp.float32)
         + brel_ref[...]
         + jnp.dot(h1_ref[:, 0:4], wroot_ref[...],
                   preferred_element_type=jnp.float32))
    h = _gelu(h)
    h_ref[...] = h
    pw = pw_ref[...]
    norm = jnp.sqrt(jnp.sum(pw * pw))
    hv = jnp.dot(h, pw.reshape(32, 1), preferred_element_type=jnp.float32)
    score = jnp.tanh(hv / norm)
    score_ref[...] = score
    key = _sort_key(score)
    row = lax.broadcasted_iota(jnp.int32, (NP, 1), 0)
    key_ref[...] = jnp.where(row < N, key, jnp.int32(-1070000000))


def _tc2(agg_partials, h1, w_rel2, b_rel2, w_root2, pool_w1):
    return pl.pallas_call(
        _tc2_body,
        out_shape=(jax.ShapeDtypeStruct((NP, 32), jnp.float32),
                   jax.ShapeDtypeStruct((NP, 1), jnp.float32),
                   jax.ShapeDtypeStruct((NP, 1), jnp.int32)),
    )(agg_partials, h1, w_rel2, b_rel2, w_root2, pool_w1)


def _sel_body(k_ref, o_ref):
    k2 = k_ref[...]
    def step(_, lohi):
        lo, hi = lohi
        mid = lo + (hi - lo) // 2
        c = jnp.sum((k2 > mid).astype(jnp.int32))
        ge = c >= K1
        return jnp.where(ge, mid, lo), jnp.where(ge, hi, mid)

    lo0 = jnp.int32(-1065353300)
    hi0 = jnp.int32(1065353300)
    _, tau = lax.fori_loop(0, 31, step, (lo0, hi0))
    quota = K1 - jnp.sum((k2 > tau).astype(jnp.int32))
    eqf = (k2 == tau).astype(jnp.float32)
    c_i = lax.broadcasted_iota(jnp.int32, (128, 128), 0)
    c_j = lax.broadcasted_iota(jnp.int32, (128, 128), 1)
    ut = (c_i < c_j).astype(jnp.float32)
    in_row = jnp.dot(eqf, ut, preferred_element_type=jnp.float32)
    r_i = lax.broadcasted_iota(jnp.int32, (80, 80), 0)
    r_j = lax.broadcasted_iota(jnp.int32, (80, 80), 1)
    lt = (r_i > r_j).astype(jnp.float32)
    rowsum = jnp.sum(eqf, axis=1, keepdims=True)
    prev_rows = jnp.dot(lt, rowsum, preferred_element_type=jnp.float32)
    tie_ord = in_row + prev_rows
    kept = (k2 > tau) | ((k2 == tau) & (tie_ord < quota.astype(jnp.float32)))
    o_ref[...] = kept.astype(jnp.float32)


def _sel(keys2d):
    return pl.pallas_call(
        _sel_body,
        out_shape=jax.ShapeDtypeStruct((NP // 128, 128), jnp.float32),
    )(keys2d)


def _rank_body(kt_ref, kf_ref, o_ref):
    r = pl.program_id(0)
    ki = kt_ref[...].reshape(128, 1)
    kj = kf_ref[...]
    i_idx = r * 128 + lax.broadcasted_iota(jnp.int32, (128, 1), 0)
    j_idx = lax.broadcasted_iota(jnp.int32, (1, NP), 1)
    t = jnp.where(j_idx < i_idx, kj + 1, kj)
    cnt = jnp.sum((t > ki).astype(jnp.int32), axis=1, keepdims=True)
    o_ref[...] = cnt.reshape(1, 128, 1)


def _ranks(keys_3d, keys_flat):
    return pl.pallas_call(
        _rank_body,
        grid=(NP // 128,),
        in_specs=[
            pl.BlockSpec((1, 128, 1), lambda r: (r, 0, 0)),
            pl.BlockSpec((1, NP), lambda r: (0, 0)),
        ],
        out_specs=pl.BlockSpec((1, 128, 1), lambda r: (r, 0, 0)),
        out_shape=jax.ShapeDtypeStruct((NP // 128, 128, 1), jnp.int32),
    )(keys_3d, keys_flat)


def _tc3_body(h_ref, score_ref, kept_ref, o_ref):
    o_ref[...] = h_ref[...] * score_ref[...] * kept_ref[...]


def _tc3(h, score, kept):
    return pl.pallas_call(
        _tc3_body,
        out_shape=jax.ShapeDtypeStruct((NP, 32), jnp.float32),
    )(h, score, kept)


def _tc4_body(aggp_ref, hp_ref, rank_ref, kept_ref, wrel_ref, brel_ref,
              wroot_ref, wout_ref, bout_ref, o_ref):
    agg2 = aggp_ref[:NP, :] + aggp_ref[NP:, :]
    z = (jnp.dot(agg2, wrel_ref[...], preferred_element_type=jnp.float32)
         + brel_ref[...]
         + jnp.dot(hp_ref[...], wroot_ref[...],
                   preferred_element_type=jnp.float32))
    h2 = _gelu(z)
    r = rank_ref[...]
    kept = kept_ref[...]
    m = lax.rem(r, jnp.int32(3))
    row0 = wout_ref[0:1, :] * (1.0 / _C0)
    row1 = wout_ref[1:2, :] * (1.0 / _C1)
    row2 = wout_ref[2:3, :] * (1.0 / _C2)
    ws = jnp.where(m == 0, row0, jnp.where(m == 1, row1, row2)) * kept
    total = jnp.sum(h2 * ws) + bout_ref[0, 0]
    o_ref[...] = total.reshape(1, 1)


def _tc4(agg2_partials, h_p, ranks, kept, w_rel3, b_rel3, w_root3, w_out3,
         b_out):
    return pl.pallas_call(
        _tc4_body,
        out_shape=jax.ShapeDtypeStruct((1, 1), jnp.float32),
    )(agg2_partials, h_p, ranks, kept, w_rel3, b_rel3, w_root3, w_out3, b_out)


def _sc_scatter(feat, src_w, dst_w, zeros_nw, width):
    mesh = plsc.VectorSubcoreMesh(core_axis_name="c", subcore_axis_name="s")

    @functools.partial(
        pl.kernel,
        out_type=jax.ShapeDtypeStruct((NC * NP, width), jnp.float32),
        mesh=mesh,
        scratch_types=[
            pltpu.VMEM((C_FAST, 128), jnp.int32),
            pltpu.VMEM((C_FAST, 128), jnp.int32),
            pltpu.VMEM((GK, 128, width), jnp.float32),
            pltpu.VMEM((GK, 128, width), jnp.float32),
            pltpu.VMEM_SHARED((NP, width), jnp.float32),
            pltpu.SemaphoreType.DMA,
            pltpu.SemaphoreType.DMA,
            pltpu.SemaphoreType.DMA,
            pltpu.SemaphoreType.DMA,
        ],
        compiler_params=pltpu.CompilerParams(use_tc_tiling_on_sc=False),
    )
    def k(feat_hbm, src_hbm, dst_hbm, zeros_hbm, out_hbm,
          src_v, dst_v, bufa, bufb, acc_sh, gsa, gsb, ssa, ssb):
        cid = lax.axis_index("c")
        sid = lax.axis_index("s")
        pltpu.sync_copy(zeros_hbm.at[pl.ds(sid * RPT, RPT)],
                        acc_sh.at[pl.ds(sid * RPT, RPT)])

        def fire_g(base, buf, sem):
            for b in range(GK):
                pltpu.async_copy(feat_hbm.at[src_v.at[base + b]],
                                 buf.at[b], sem)

        def drain_g(buf, sem):
            for b in range(GK):
                pltpu.make_async_copy(feat_hbm.at[pl.ds(0, 128)],
                                      buf.at[b], sem).wait()

        def fire_s(base, buf, sem):
            for b in range(GK):
                pltpu.async_copy(buf.at[b], acc_sh.at[dst_v.at[base + b]],
                                 sem, add=True)

        def drain_s(buf, sem):
            for b in range(GK):
                pltpu.make_async_copy(buf.at[b], acc_sh.at[dst_v.at[b]],
                                      sem).wait()

        hbase = jnp.where(cid == 0, sid * C_SLOW, 16 * C_SLOW + sid * C_FAST)
        npairs = jnp.where(cid == 0, C_SLOW // (2 * GK), C_FAST // (2 * GK))
        pltpu.sync_copy(src_hbm.at[pl.ds(hbase, C_FAST)], src_v)
        pltpu.sync_copy(dst_hbm.at[pl.ds(hbase, C_FAST)], dst_v)
        plsc.subcore_barrier()

        fire_g(0, bufa, gsa)

        def body(kk, carry):
            ja = kk * (2 * GK)
            jb = ja + GK
            drain_g(bufa, gsa)
            fire_s(ja, bufa, ssa)

            @pl.when(kk > 0)
            def _():
                drain_s(bufb, ssb)

            fire_g(jb, bufb, gsb)
            drain_g(bufb, gsb)
            fire_s(jb, bufb, ssb)
            drain_s(bufa, ssa)

            @pl.when(kk < npairs - 1)
            def _():
                fire_g(ja + 2 * GK, bufa, gsa)

            return carry

        lax.fori_loop(0, npairs, body, 0)
        drain_s(bufb, ssb)

        plsc.subcore_barrier()
        pltpu.sync_copy(acc_sh.at[pl.ds(sid * RPT, RPT)],
                        out_hbm.at[pl.ds(cid * NP + sid * RPT, RPT)])

    return k(feat, src_w, dst_w, zeros_nw)


def kernel(x, edge_index, W_rel1, b_rel1, W_root1, W_rel2, b_rel2, W_root2,
           pool_w1, W_rel3, b_rel3, W_root3, W_out, b_out):
    f32 = jnp.float32
    src = edge_index[0]
    dst = edge_index[1]
    pad = jnp.full((EP - E,), N, jnp.int32)
    src_w = jnp.concatenate([src, pad]).reshape(NW * CHUNKS, 128)
    dst_w = jnp.concatenate([dst, pad]).reshape(NW * CHUNKS, 128)
    zeros16 = jnp.zeros((NP, 16), f32)
    zeros32 = jnp.zeros((NP, 32), f32)
    b_rel2r = b_rel2.reshape(1, 32)
    b_rel3r = b_rel3.reshape(1, 64)
    pool_r = pool_w1.reshape(1, 32)
    w_out3 = W_out.reshape(3, 64)
    b_outr = b_out.reshape(1, 1)

    h1 = _tc1(x, W_root1, b_rel1.reshape(1, 4))

    aggp1 = _sc_scatter(h1, src_w, dst_w, zeros16, 16)

    h, score, keys = _tc2(aggp1, h1, W_rel2, b_rel2r, W_root2, pool_r)

    kept = _sel(keys.reshape(NP // 128, 128)).reshape(NP, 1)
    h_p = _tc3(h, score, kept)

    aggp2 = _sc_scatter(h_p, src_w, dst_w, zeros32, 32)

    keys_3d = keys.reshape(NP // 128, 128, 1)
    keys_flat = keys.reshape(1, NP)
    ranks = _ranks(keys_3d, keys_flat).reshape(NP, 1)

    out = _tc4(aggp2, h_p, ranks, kept, W_rel3, b_rel3r, W_root3, w_out3,
               b_outr)
    return out.reshape(1)


# speedup vs baseline: 1.0116x; 1.0116x over previous
import functools

import jax
import jax.numpy as jnp
from jax import lax
from jax.experimental import pallas as pl
from jax.experimental.pallas import tpu as pltpu
from jax.experimental.pallas import tpu_sc as plsc

N = 10000
NP = 10240
E = 320000
EP = 327680
K1 = 7000
NW = 32
NC = 2
NS = 16
CHUNKS = 80
GK = 8
C_SLOW = 48
C_FAST = 112
RPT = NP // NS

_C0 = (K1 + 2) // 3
_C1 = (K1 + 1) // 3
_C2 = K1 // 3


def _gelu(v):
    return 0.5 * v * (1.0 + lax.erf(v * 0.7071067811865476))


def _sort_key(score):
    s = lax.bitcast_convert_type(score, jnp.int32)
    return jnp.where(s >= 0, s, s ^ jnp.int32(0x7FFFFFFF))


def _tc1_body(x_ref, w_ref, b_ref, o_ref):
    h = jnp.dot(x_ref[...], w_ref[...],
                preferred_element_type=jnp.float32) + b_ref[...]
    o_ref[...] = jnp.zeros((NP, 16), jnp.float32)
    o_ref[:N, 0:4] = _gelu(h)


def _tc1(x, w_root1, b_rel1):
    return pl.pallas_call(
        _tc1_body,
        out_shape=jax.ShapeDtypeStruct((NP, 16), jnp.float32),
    )(x, w_root1, b_rel1)


def _tc2_body(aggp_ref, h1_ref, wrel_ref, brel_ref, wroot_ref, pw_ref,
              h_ref, score_ref, key_ref):
    agg = aggp_ref[:NP, 0:4] + aggp_ref[NP:, 0:4]
    h = (jnp.dot(agg, wrel_ref[...], preferred_element_type=jnp.float32)
         + brel_ref[...]
         + jnp.dot(h1_ref[:, 0:4], wroot_ref[...],
                   preferred_element_type=jnp.float32))
    h = _gelu(h)
    h_ref[...] = h
    pw = pw_ref[...]
    norm = jnp.sqrt(jnp.sum(pw * pw))
    hv = jnp.dot(h, pw.reshape(32, 1), preferred_element_type=jnp.float32)
    score = jnp.tanh(hv / norm)
    score_ref[...] = score
    key = _sort_key(score)
    row = lax.broadcasted_iota(jnp.int32, (NP, 1), 0)
    key_ref[...] = jnp.where(row < N, key, jnp.int32(-1070000000))


def _tc2(agg_partials, h1, w_rel2, b_rel2, w_root2, pool_w1):
    return pl.pallas_call(
        _tc2_body,
        out_shape=(jax.ShapeDtypeStruct((NP, 32), jnp.float32),
                   jax.ShapeDtypeStruct((NP, 1), jnp.float32),
                   jax.ShapeDtypeStruct((NP, 1), jnp.int32)),
    )(agg_partials, h1, w_rel2, b_rel2, w_root2, pool_w1)


def _sel_body(k_ref, o_ref):
    k2 = k_ref[...]
    def step(_, lohi):
        lo, hi = lohi
        mid = lo + (hi - lo) // 2
        c = jnp.sum((k2 > mid).astype(jnp.int32))
        ge = c >= K1
        return jnp.where(ge, mid, lo), jnp.where(ge, hi, mid)

    lo0 = jnp.int32(-1065353300)
    hi0 = jnp.int32(1065353300)
    _, tau = lax.fori_loop(0, 31, step, (lo0, hi0))
    quota = K1 - jnp.sum((k2 > tau).astype(jnp.int32))
    eqf = (k2 == tau).astype(jnp.float32)
    c_i = lax.broadcasted_iota(jnp.int32, (128, 128), 0)
    c_j = lax.broadcasted_iota(jnp.int32, (128, 128), 1)
    ut = (c_i < c_j).astype(jnp.float32)
    in_row = jnp.dot(eqf, ut, preferred_element_type=jnp.float32)
    r_i = lax.broadcasted_iota(jnp.int32, (80, 80), 0)
    r_j = lax.broadcasted_iota(jnp.int32, (80, 80), 1)
    lt = (r_i > r_j).astype(jnp.float32)
    rowsum = jnp.sum(eqf, axis=1, keepdims=True)
    prev_rows = jnp.dot(lt, rowsum, preferred_element_type=jnp.float32)
    tie_ord = in_row + prev_rows
    kept = (k2 > tau) | ((k2 == tau) & (tie_ord < quota.astype(jnp.float32)))
    o_ref[...] = kept.astype(jnp.float32)


def _sel(keys2d):
    return pl.pallas_call(
        _sel_body,
        out_shape=jax.ShapeDtypeStruct((NP // 128, 128), jnp.float32),
    )(keys2d)


def _rank_body(kt_ref, kf_ref, o_ref):
    r = pl.program_id(0)
    ki = kt_ref[...].reshape(128, 1)
    kj = kf_ref[...]
    i_idx = r * 128 + lax.broadcasted_iota(jnp.int32, (128, 1), 0)
    j_idx = lax.broadcasted_iota(jnp.int32, (1, NP), 1)
    t = jnp.where(j_idx < i_idx, kj + 1, kj)
    cnt = jnp.sum((t > ki).astype(jnp.int32), axis=1, keepdims=True)
    o_ref[...] = cnt.reshape(1, 128, 1)


def _ranks(keys_3d, keys_flat):
    return pl.pallas_call(
        _rank_body,
        grid=(NP // 128,),
        in_specs=[
            pl.BlockSpec((1, 128, 1), lambda r: (r, 0, 0)),
            pl.BlockSpec((1, NP), lambda r: (0, 0)),
        ],
        out_specs=pl.BlockSpec((1, 128, 1), lambda r: (r, 0, 0)),
        out_shape=jax.ShapeDtypeStruct((NP // 128, 128, 1), jnp.int32),
    )(keys_3d, keys_flat)


def _tc3_body(h_ref, score_ref, kept_ref, o_ref):
    o_ref[...] = h_ref[...] * score_ref[...] * kept_ref[...]


def _tc3(h, score, kept):
    return pl.pallas_call(
        _tc3_body,
        out_shape=jax.ShapeDtypeStruct((NP, 32), jnp.float32),
    )(h, score, kept)


def _tc4_body(aggp_ref, hp_ref, rank_ref, kept_ref, wrel_ref, brel_ref,
              wroot_ref, wout_ref, bout_ref, o_ref):
    agg2 = aggp_ref[:NP, :] + aggp_ref[NP:, :]
    z = (jnp.dot(agg2, wrel_ref[...], preferred_element_type=jnp.float32)
         + brel_ref[...]
         + jnp.dot(hp_ref[...], wroot_ref[...],
                   preferred_element_type=jnp.float32))
    h2 = _gelu(z)
    r = rank_ref[...]
    kept = kept_ref[...]
    m = lax.rem(r, jnp.int32(3))
    row0 = wout_ref[0:1, :] * (1.0 / _C0)
    row1 = wout_ref[1:2, :] * (1.0 / _C1)
    row2 = wout_ref[2:3, :] * (1.0 / _C2)
    ws = jnp.where(m == 0, row0, jnp.where(m == 1, row1, row2)) * kept
    total = jnp.sum(h2 * ws) + bout_ref[0, 0]
    o_ref[...] = total.reshape(1, 1)


def _tc4(agg2_partials, h_p, ranks, kept, w_rel3, b_rel3, w_root3, w_out3,
         b_out):
    return pl.pallas_call(
        _tc4_body,
        out_shape=jax.ShapeDtypeStruct((1, 1), jnp.float32),
    )(agg2_partials, h_p, ranks, kept, w_rel3, b_rel3, w_root3, w_out3, b_out)


def _sc_scatter(feat, src_w, dst_w, zeros_nw, width):
    mesh = plsc.VectorSubcoreMesh(core_axis_name="c", subcore_axis_name="s")

    @functools.partial(
        pl.kernel,
        out_type=jax.ShapeDtypeStruct((NC * NP, width), jnp.float32),
        mesh=mesh,
        scratch_types=[
            pltpu.VMEM((C_FAST, 128), jnp.int32),
            pltpu.VMEM((C_FAST, 128), jnp.int32),
            pltpu.VMEM((GK, 128, width), jnp.float32),
            pltpu.VMEM((GK, 128, width), jnp.float32),
            pltpu.VMEM_SHARED((NP, width), jnp.float32),
            pltpu.SemaphoreType.DMA,
            pltpu.SemaphoreType.DMA,
            pltpu.SemaphoreType.DMA,
            pltpu.SemaphoreType.DMA,
        ],
        compiler_params=pltpu.CompilerParams(use_tc_tiling_on_sc=False),
    )
    def k(feat_hbm, src_hbm, dst_hbm, zeros_hbm, out_hbm,
          src_v, dst_v, bufa, bufb, acc_sh, gsa, gsb, ssa, ssb):
        cid = lax.axis_index("c")
        sid = lax.axis_index("s")
        pltpu.sync_copy(zeros_hbm.at[pl.ds(sid * RPT, RPT)],
                        acc_sh.at[pl.ds(sid * RPT, RPT)])

        def fire_g(base, buf, sem):
            for b in range(GK):
                pltpu.async_copy(feat_hbm.at[src_v.at[base + b]],
                                 buf.at[b], sem)

        def drain_g(buf, sem):
            for b in range(GK):
                pltpu.make_async_copy(feat_hbm.at[pl.ds(0, 128)],
                                      buf.at[b], sem).wait()

        def fire_s(base, buf, sem):
            for b in range(GK):
                pltpu.async_copy(buf.at[b], acc_sh.at[dst_v.at[base + b]],
                                 sem, add=True)

        def drain_s(buf, sem):
            for b in range(GK):
                pltpu.make_async_copy(buf.at[b], acc_sh.at[dst_v.at[b]],
                                      sem).wait()

        hbase = jnp.where(cid == 1, sid * C_SLOW, 16 * C_SLOW + sid * C_FAST)
        npairs = jnp.where(cid == 1, C_SLOW // (2 * GK), C_FAST // (2 * GK))
        pltpu.sync_copy(src_hbm.at[pl.ds(hbase, C_FAST)], src_v)
        pltpu.sync_copy(dst_hbm.at[pl.ds(hbase, C_FAST)], dst_v)
        plsc.subcore_barrier()

        fire_g(0, bufa, gsa)

        def body(kk, carry):
            ja = kk * (2 * GK)
            jb = ja + GK
            drain_g(bufa, gsa)
            fire_s(ja, bufa, ssa)

            @pl.when(kk > 0)
            def _():
                drain_s(bufb, ssb)

            fire_g(jb, bufb, gsb)
            drain_g(bufb, gsb)
            fire_s(jb, bufb, ssb)
            drain_s(bufa, ssa)

            @pl.when(kk < npairs - 1)
            def _():
                fire_g(ja + 2 * GK, bufa, gsa)

            return carry

        lax.fori_loop(0, npairs, body, 0)
        drain_s(bufb, ssb)

        plsc.subcore_barrier()
        pltpu.sync_copy(acc_sh.at[pl.ds(sid * RPT, RPT)],
                        out_hbm.at[pl.ds(cid * NP + sid * RPT, RPT)])

    return k(feat, src_w, dst_w, zeros_nw)


def kernel(x, edge_index, W_rel1, b_rel1, W_root1, W_rel2, b_rel2, W_root2,
           pool_w1, W_rel3, b_rel3, W_root3, W_out, b_out):
    f32 = jnp.float32
    src = edge_index[0]
    dst = edge_index[1]
    pad = jnp.full((EP - E,), N, jnp.int32)
    src_w = jnp.concatenate([src, pad]).reshape(NW * CHUNKS, 128)
    dst_w = jnp.concatenate([dst, pad]).reshape(NW * CHUNKS, 128)
    zeros16 = jnp.zeros((NP, 16), f32)
    zeros32 = jnp.zeros((NP, 32), f32)
    b_rel2r = b_rel2.reshape(1, 32)
    b_rel3r = b_rel3.reshape(1, 64)
    pool_r = pool_w1.reshape(1, 32)
    w_out3 = W_out.reshape(3, 64)
    b_outr = b_out.reshape(1, 1)

    h1 = _tc1(x, W_root1, b_rel1.reshape(1, 4))

    aggp1 = _sc_scatter(h1, src_w, dst_w, zeros16, 16)

    h, score, keys = _tc2(aggp1, h1, W_rel2, b_rel2r, W_root2, pool_r)

    kept = _sel(keys.reshape(NP // 128, 128)).reshape(NP, 1)
    h_p = _tc3(h, score, kept)

    aggp2 = _sc_scatter(h_p, src_w, dst_w, zeros32, 32)

    keys_3d = keys.reshape(NP // 128, 128, 1)
    keys_flat = keys.reshape(1, NP)
    ranks = _ranks(keys_3d, keys_flat).reshape(NP, 1)

    out = _tc4(aggp2, h_p, ranks, kept, W_rel3, b_rel3r, W_root3, w_out3,
               b_outr)
    return out.reshape(1)


# speedup vs baseline: 1.0591x; 1.0470x over previous
import functools

import jax
import jax.numpy as jnp
from jax import lax
from jax.experimental import pallas as pl
from jax.experimental.pallas import tpu as pltpu
from jax.experimental.pallas import tpu_sc as plsc

N = 10000
NP = 10240
E = 320000
EP = 327680
K1 = 7000
NW = 32
NC = 2
NS = 16
CHUNKS = 80
GK = 10
C_SLOW = 80
C_FAST = 80
RPT = NP // NS

_C0 = (K1 + 2) // 3
_C1 = (K1 + 1) // 3
_C2 = K1 // 3


def _gelu(v):
    return 0.5 * v * (1.0 + lax.erf(v * 0.7071067811865476))


def _sort_key(score):
    s = lax.bitcast_convert_type(score, jnp.int32)
    return jnp.where(s >= 0, s, s ^ jnp.int32(0x7FFFFFFF))


def _tc1_body(x_ref, w_ref, b_ref, o_ref):
    h = jnp.dot(x_ref[...], w_ref[...],
                preferred_element_type=jnp.float32) + b_ref[...]
    o_ref[...] = jnp.zeros((NP, 16), jnp.float32)
    o_ref[:N, 0:4] = _gelu(h)


def _tc1(x, w_root1, b_rel1):
    return pl.pallas_call(
        _tc1_body,
        out_shape=jax.ShapeDtypeStruct((NP, 16), jnp.float32),
    )(x, w_root1, b_rel1)


def _tc2_body(aggp_ref, h1_ref, wrel_ref, brel_ref, wroot_ref, pw_ref,
              h_ref, score_ref, key_ref):
    agg = aggp_ref[:NP, 0:4] + aggp_ref[NP:, 0:4]
    h = (jnp.dot(agg, wrel_ref[...], preferred_element_type=jnp.float32)
         + brel_ref[...]
         + jnp.dot(h1_ref[:, 0:4], wroot_ref[...],
                   preferred_element_type=jnp.float32))
    h = _gelu(h)
    h_ref[...] = h
    pw = pw_ref[...]
    norm = jnp.sqrt(jnp.sum(pw * pw))
    hv = jnp.dot(h, pw.reshape(32, 1), preferred_element_type=jnp.float32)
    score = jnp.tanh(hv / norm)
    score_ref[...] = score
    key = _sort_key(score)
    row = lax.broadcasted_iota(jnp.int32, (NP, 1), 0)
    key_ref[...] = jnp.where(row < N, key, jnp.int32(-1070000000))


def _tc2(agg_partials, h1, w_rel2, b_rel2, w_root2, pool_w1):
    return pl.pallas_call(
        _tc2_body,
        out_shape=(jax.ShapeDtypeStruct((NP, 32), jnp.float32),
                   jax.ShapeDtypeStruct((NP, 1), jnp.float32),
                   jax.ShapeDtypeStruct((NP, 1), jnp.int32)),
    )(agg_partials, h1, w_rel2, b_rel2, w_root2, pool_w1)


def _sel_body(k_ref, o_ref):
    k2 = k_ref[...]
    def step(_, lohi):
        lo, hi = lohi
        mid = lo + (hi - lo) // 2
        c = jnp.sum((k2 > mid).astype(jnp.int32))
        ge = c >= K1
        return jnp.where(ge, mid, lo), jnp.where(ge, hi, mid)

    lo0 = jnp.int32(-1065353300)
    hi0 = jnp.int32(1065353300)
    _, tau = lax.fori_loop(0, 31, step, (lo0, hi0))
    quota = K1 - jnp.sum((k2 > tau).astype(jnp.int32))
    eqf = (k2 == tau).astype(jnp.float32)
    c_i = lax.broadcasted_iota(jnp.int32, (128, 128), 0)
    c_j = lax.broadcasted_iota(jnp.int32, (128, 128), 1)
    ut = (c_i < c_j).astype(jnp.float32)
    in_row = jnp.dot(eqf, ut, preferred_element_type=jnp.float32)
    r_i = lax.broadcasted_iota(jnp.int32, (80, 80), 0)
    r_j = lax.broadcasted_iota(jnp.int32, (80, 80), 1)
    lt = (r_i > r_j).astype(jnp.float32)
    rowsum = jnp.sum(eqf, axis=1, keepdims=True)
    prev_rows = jnp.dot(lt, rowsum, preferred_element_type=jnp.float32)
    tie_ord = in_row + prev_rows
    kept = (k2 > tau) | ((k2 == tau) & (tie_ord < quota.astype(jnp.float32)))
    o_ref[...] = kept.astype(jnp.float32)


def _sel(keys2d):
    return pl.pallas_call(
        _sel_body,
        out_shape=jax.ShapeDtypeStruct((NP // 128, 128), jnp.float32),
    )(keys2d)


def _rank_body(kt_ref, kf_ref, o_ref):
    r = pl.program_id(0)
    ki = kt_ref[...].reshape(128, 1)
    kj = kf_ref[...]
    i_idx = r * 128 + lax.broadcasted_iota(jnp.int32, (128, 1), 0)
    j_idx = lax.broadcasted_iota(jnp.int32, (1, NP), 1)
    t = jnp.where(j_idx < i_idx, kj + 1, kj)
    cnt = jnp.sum((t > ki).astype(jnp.int32), axis=1, keepdims=True)
    o_ref[...] = cnt.reshape(1, 128, 1)


def _ranks(keys_3d, keys_flat):
    return pl.pallas_call(
        _rank_body,
        grid=(NP // 128,),
        in_specs=[
            pl.BlockSpec((1, 128, 1), lambda r: (r, 0, 0)),
            pl.BlockSpec((1, NP), lambda r: (0, 0)),
        ],
        out_specs=pl.BlockSpec((1, 128, 1), lambda r: (r, 0, 0)),
        out_shape=jax.ShapeDtypeStruct((NP // 128, 128, 1), jnp.int32),
    )(keys_3d, keys_flat)


def _tc3_body(h_ref, score_ref, kept_ref, o_ref):
    o_ref[...] = h_ref[...] * score_ref[...] * kept_ref[...]


def _tc3(h, score, kept):
    return pl.pallas_call(
        _tc3_body,
        out_shape=jax.ShapeDtypeStruct((NP, 32), jnp.float32),
    )(h, score, kept)


def _tc4_body(aggp_ref, hp_ref, rank_ref, kept_ref, wrel_ref, brel_ref,
              wroot_ref, wout_ref, bout_ref, o_ref):
    agg2 = aggp_ref[:NP, :] + aggp_ref[NP:, :]
    z = (jnp.dot(agg2, wrel_ref[...], preferred_element_type=jnp.float32)
         + brel_ref[...]
         + jnp.dot(hp_ref[...], wroot_ref[...],
                   preferred_element_type=jnp.float32))
    h2 = _gelu(z)
    r = rank_ref[...]
    kept = kept_ref[...]
    m = lax.rem(r, jnp.int32(3))
    row0 = wout_ref[0:1, :] * (1.0 / _C0)
    row1 = wout_ref[1:2, :] * (1.0 / _C1)
    row2 = wout_ref[2:3, :] * (1.0 / _C2)
    ws = jnp.where(m == 0, row0, jnp.where(m == 1, row1, row2)) * kept
    total = jnp.sum(h2 * ws) + bout_ref[0, 0]
    o_ref[...] = total.reshape(1, 1)


def _tc4(agg2_partials, h_p, ranks, kept, w_rel3, b_rel3, w_root3, w_out3,
         b_out):
    return pl.pallas_call(
        _tc4_body,
        out_shape=jax.ShapeDtypeStruct((1, 1), jnp.float32),
    )(agg2_partials, h_p, ranks, kept, w_rel3, b_rel3, w_root3, w_out3, b_out)


def _sc_scatter(feat, src_w, dst_w, zeros_nw, width):
    mesh = plsc.VectorSubcoreMesh(core_axis_name="c", subcore_axis_name="s")

    @functools.partial(
        pl.kernel,
        out_type=jax.ShapeDtypeStruct((NC * NP, width), jnp.float32),
        mesh=mesh,
        scratch_types=[
            pltpu.VMEM((C_FAST, 128), jnp.int32),
            pltpu.VMEM((C_FAST, 128), jnp.int32),
            pltpu.VMEM((GK, 128, width), jnp.float32),
            pltpu.VMEM((GK, 128, width), jnp.float32),
            pltpu.VMEM_SHARED((NP, width), jnp.float32),
            pltpu.SemaphoreType.DMA,
            pltpu.SemaphoreType.DMA,
            pltpu.SemaphoreType.DMA,
            pltpu.SemaphoreType.DMA,
        ],
        compiler_params=pltpu.CompilerParams(use_tc_tiling_on_sc=False),
    )
    def k(feat_hbm, src_hbm, dst_hbm, zeros_hbm, out_hbm,
          src_v, dst_v, bufa, bufb, acc_sh, gsa, gsb, ssa, ssb):
        cid = lax.axis_index("c")
        sid = lax.axis_index("s")
        pltpu.sync_copy(zeros_hbm.at[pl.ds(sid * RPT, RPT)],
                        acc_sh.at[pl.ds(sid * RPT, RPT)])

        def fire_g(base, buf, sem):
            for b in range(GK):
                pltpu.async_copy(feat_hbm.at[src_v.at[base + b]],
                                 buf.at[b], sem)

        def drain_g(buf, sem):
            for b in range(GK):
                pltpu.make_async_copy(feat_hbm.at[pl.ds(0, 128)],
                                      buf.at[b], sem).wait()

        def fire_s(base, buf, sem):
            for b in range(GK):
                pltpu.async_copy(buf.at[b], acc_sh.at[dst_v.at[base + b]],
                                 sem, add=True)

        def drain_s(buf, sem):
            for b in range(GK):
                pltpu.make_async_copy(buf.at[b], acc_sh.at[dst_v.at[b]],
                                      sem).wait()

        hbase = jnp.where(cid == 1, sid * C_SLOW, 16 * C_SLOW + sid * C_FAST)
        npairs = jnp.where(cid == 1, C_SLOW // (2 * GK), C_FAST // (2 * GK))
        pltpu.sync_copy(src_hbm.at[pl.ds(hbase, C_FAST)], src_v)
        pltpu.sync_copy(dst_hbm.at[pl.ds(hbase, C_FAST)], dst_v)
        plsc.subcore_barrier()

        fire_g(0, bufa, gsa)

        def body(kk, carry):
            ja = kk * (2 * GK)
            jb = ja + GK
            drain_g(bufa, gsa)
            fire_s(ja, bufa, ssa)

            @pl.when(kk > 0)
            def _():
                drain_s(bufb, ssb)

            fire_g(jb, bufb, gsb)
            drain_g(bufb, gsb)
            fire_s(jb, bufb, ssb)
            drain_s(bufa, ssa)

            @pl.when(kk < npairs - 1)
            def _():
                fire_g(ja + 2 * GK, bufa, gsa)

            return carry

        lax.fori_loop(0, npairs, body, 0)
        drain_s(bufb, ssb)

        plsc.subcore_barrier()
        pltpu.sync_copy(acc_sh.at[pl.ds(sid * RPT, RPT)],
                        out_hbm.at[pl.ds(cid * NP + sid * RPT, RPT)])

    return k(feat, src_w, dst_w, zeros_nw)


def kernel(x, edge_index, W_rel1, b_rel1, W_root1, W_rel2, b_rel2, W_root2,
           pool_w1, W_rel3, b_rel3, W_root3, W_out, b_out):
    f32 = jnp.float32
    src = edge_index[0]
    dst = edge_index[1]
    pad = jnp.full((EP - E,), N, jnp.int32)
    src_w = jnp.concatenate([src, pad]).reshape(NW * CHUNKS, 128)
    dst_w = jnp.concatenate([dst, pad]).reshape(NW * CHUNKS, 128)
    zeros16 = jnp.zeros((NP, 16), f32)
    zeros32 = jnp.zeros((NP, 32), f32)
    b_rel2r = b_rel2.reshape(1, 32)
    b_rel3r = b_rel3.reshape(1, 64)
    pool_r = pool_w1.reshape(1, 32)
    w_out3 = W_out.reshape(3, 64)
    b_outr = b_out.reshape(1, 1)

    h1 = _tc1(x, W_root1, b_rel1.reshape(1, 4))

    aggp1 = _sc_scatter(h1, src_w, dst_w, zeros16, 16)

    h, score, keys = _tc2(aggp1, h1, W_rel2, b_rel2r, W_root2, pool_r)

    kept = _sel(keys.reshape(NP // 128, 128)).reshape(NP, 1)
    h_p = _tc3(h, score, kept)

    aggp2 = _sc_scatter(h_p, src_w, dst_w, zeros32, 32)

    keys_3d = keys.reshape(NP // 128, 128, 1)
    keys_flat = keys.reshape(1, NP)
    ranks = _ranks(keys_3d, keys_flat).reshape(NP, 1)

    out = _tc4(aggp2, h_p, ranks, kept, W_rel3, b_rel3r, W_root3, w_out3,
               b_outr)
    return out.reshape(1)
